# baseline (speedup 1.0000x reference)
import functools

import jax
import jax.numpy as jnp
from jax.experimental import pallas as pl
from jax.experimental.pallas import tpu as pltpu

_HUFFMAN_PATHS = ((0, 1), (1, 0), (0, 0, 1), (1, 1, 0))
_NCOL = 16


def _body(h_ref, tw_ref, w_ref, out_ref):
    h = h_ref[...]
    bm = h.shape[0]
    tw = tw_ref[...]
    n = pl.num_programs(0) * bm
    x = jax.lax.dot_general(
        h,
        w_ref[...],
        (((1,), (1,)), ((), ())),
        preferred_element_type=jnp.float32,
    )
    soft = jnp.maximum(x, 0.0) + jnp.log1p(jnp.exp(-jnp.abs(x)))
    total = jnp.float32(0.0)
    off = 0
    for w, path in enumerate(_HUFFMAN_PATHS):
        lw = len(path)
        ones = [off + j for j, b in enumerate(path) if b == 1]
        lo, hi = ones[0], ones[-1] + 1
        soft_w = jnp.sum(soft[:, off : off + lw], axis=1, keepdims=True)
        xs_w = jnp.sum(x[:, lo:hi], axis=1, keepdims=True)
        per_row = (soft_w - xs_w) * (1.0 / lw)
        sel = (tw == w).astype(jnp.float32)
        total = total + jnp.sum(sel * per_row)
        off += lw

    @pl.when(pl.program_id(0) == 0)
    def _():
        out_ref[0, 0] = 0.0

    out_ref[0, 0] += total / jnp.float32(n)


@functools.partial(jax.jit, static_argnames=("interpret", "bm"))
def kernel(hidden, target_words, W_0, W_1, W_2, W_3, interpret=False, bm=512):
    batch, hdim = hidden.shape
    grid = batch // bm

    wstack = jnp.concatenate([W_0, W_1, W_2, W_3], axis=0)
    wstack = jnp.pad(wstack, ((0, _NCOL - wstack.shape[0]), (0, 0)))
    tw2d = target_words.astype(jnp.int32).reshape(batch, 1)

    out = pl.pallas_call(
        _body,
        grid=(grid,),
        in_specs=[
            pl.BlockSpec((bm, hdim), lambda i: (i, 0)),
            pl.BlockSpec((bm, 1), lambda i: (i, 0)),
            pl.BlockSpec((_NCOL, hdim), lambda i: (0, 0)),
        ],
        out_specs=pl.BlockSpec(
            (1, 1), lambda i: (0, 0), memory_space=pltpu.SMEM
        ),
        out_shape=jax.ShapeDtypeStruct((1, 1), jnp.float32),
        interpret=interpret,
    )(hidden, tw2d, wstack)
    return out[0, 0]
